# baseline (speedup 1.0000x reference)
import functools

import jax
import jax.numpy as jnp
from jax import lax
from jax.experimental import pallas as pl
from jax.experimental.pallas import tpu as pltpu
from jax.experimental.pallas import tpu_sc as plsc

_IDX_CHUNK = 128
_L = 16


def kernel(x, table):
    B = x.shape[0]
    V, D = table.shape
    pack = 128 // D
    info = plsc.get_sparse_core_info()
    NC, NS = info.num_cores, info.num_subcores
    NW = NC * NS
    b_per_w = B // NW
    n_chunks = b_per_w // _IDX_CHUNK
    g_per_chunk = _IDX_CHUNK // _L
    mesh = plsc.VectorSubcoreMesh(core_axis_name="c", subcore_axis_name="s")

    @functools.partial(
        pl.kernel,
        mesh=mesh,
        compiler_params=pltpu.CompilerParams(needs_layout_passes=False, use_tc_tiling_on_sc=True),
        out_type=jax.ShapeDtypeStruct((B, D), jnp.float32),
        scratch_types=[
            pltpu.VMEM((b_per_w,), jnp.int32),
            pltpu.VMEM((b_per_w,), jnp.int32),
            pltpu.VMEM((2, _IDX_CHUNK, 128), jnp.float32),
            pltpu.VMEM((b_per_w, D), jnp.float32),
            pltpu.SemaphoreType.DMA,
            pltpu.SemaphoreType.DMA,
        ],
    )
    def _emb(x_hbm, table_hbm, out_hbm, idx_v, pk_v, rp_v, out_v, sem0, sem1):
        wid = lax.axis_index("s") * NC + lax.axis_index("c")
        base = wid * b_per_w
        sems = (sem0, sem1)
        pltpu.sync_copy(x_hbm.at[pl.ds(base, b_per_w)], idx_v)
        for k in range(b_per_w // _L):
            sl = pl.ds(k * _L, _L)
            pk_v[sl] = lax.shift_right_logical(idx_v[sl], 2)

        def gather_chunk(j):
            return pltpu.async_copy(
                table_hbm.at[pk_v.at[pl.ds(j * _IDX_CHUNK, _IDX_CHUNK)]],
                rp_v.at[j % 2],
                sems[j % 2],
            )

        def extract_chunk(j):
            buf = rp_v.at[j % 2]

            def body(g, carry):
                row16 = g * _L + lax.iota(jnp.int32, _L)
                idx16 = idx_v[pl.ds(j * _IDX_CHUNK + g * _L, _L)]
                off = (idx16 & (pack - 1)) * D
                orow = j * _IDX_CHUNK + row16
                for c in range(D):
                    vals = plsc.load_gather(buf, [row16, off + c])
                    col = jnp.full((_L,), c, jnp.int32)
                    plsc.store_scatter(out_v, [orow, col], vals)
                return carry

            lax.fori_loop(0, g_per_chunk, body, 0)

        pending = gather_chunk(0)
        for j in range(n_chunks):
            pending.wait()
            if j + 1 < n_chunks:
                pending = gather_chunk(j + 1)
            extract_chunk(j)
        pltpu.sync_copy(out_v, out_hbm.at[pl.ds(base, b_per_w)])

    return _emb(x, table.reshape(V // pack, 128))
